# baseline (speedup 1.0000x reference)
import jax
import jax.numpy as jnp
from jax.experimental import pallas as pl
from jax.experimental.pallas import tpu as pltpu

NUM_EXPERTS = 8
D_MODEL = 768


def _moe_body(logits_ref, x_ref, w_ref, b_ref, out_ref,
              xb_ref, i1_ref, i2_ref, w1_ref, w2_ref):
    e = pl.program_id(0)

    @pl.when(e == 0)
    def _init():
        logits = logits_ref[...]
        m1 = jnp.max(logits, axis=-1, keepdims=True)
        i1 = jnp.argmax(logits, axis=-1)[:, None]
        masked = jnp.where(
            jax.lax.broadcasted_iota(jnp.int32, logits.shape, 1) == i1,
            jnp.full_like(logits, -jnp.inf), logits)
        m2 = jnp.max(masked, axis=-1, keepdims=True)
        i2 = jnp.argmax(masked, axis=-1)[:, None]
        e2 = jnp.exp(m2 - m1)
        i1_ref[...] = i1
        i2_ref[...] = i2
        w1_ref[...] = 1.0 / (1.0 + e2)
        w2_ref[...] = e2 / (1.0 + e2)
        xb_ref[...] = x_ref[...].astype(jnp.bfloat16)
        out_ref[...] = jnp.zeros_like(out_ref)

    ce = jnp.where(i1_ref[...] == e, w1_ref[...],
                   jnp.where(i2_ref[...] == e, w2_ref[...], 0.0))
    y = jax.lax.dot_general(
        xb_ref[...], w_ref[0].astype(jnp.bfloat16), (((1,), (0,)), ((), ())),
        preferred_element_type=jnp.float32,
    ) + b_ref[0]
    out_ref[...] += ce * y


def kernel(hidden, gate_logits, W_experts, b_experts):
    T, D = hidden.shape
    return pl.pallas_call(
        _moe_body,
        grid=(NUM_EXPERTS,),
        in_specs=[
            pl.BlockSpec((T, NUM_EXPERTS), lambda e: (0, 0)),
            pl.BlockSpec((T, D), lambda e: (0, 0)),
            pl.BlockSpec((1, D, D), lambda e: (e, 0, 0)),
            pl.BlockSpec((1, 1, D), lambda e: (e, 0, 0)),
        ],
        out_specs=pl.BlockSpec((T, D), lambda e: (0, 0)),
        out_shape=jax.ShapeDtypeStruct((T, D), jnp.float32),
        scratch_shapes=[
            pltpu.VMEM((T, D), jnp.bfloat16),
            pltpu.VMEM((T, 1), jnp.int32),
            pltpu.VMEM((T, 1), jnp.int32),
            pltpu.VMEM((T, 1), jnp.float32),
            pltpu.VMEM((T, 1), jnp.float32),
        ],
    )(gate_logits, hidden, W_experts, b_experts.reshape(NUM_EXPERTS, 1, D))
